# baseline (speedup 1.0000x reference)
import functools

import jax
import jax.numpy as jnp
from jax.experimental import pallas as pl

B = 2048
X_SIZE = 128
H_SIZE = 128
OUT_SIZE = 128
NUM_EMB = 16


def _expert_step(x_ref, idx_ref, w1_ref, b1_ref, w2_ref, b2_ref, out_ref):
    e = pl.program_id(0)

    @pl.when(e == 0)
    def _init():
        out_ref[...] = jnp.zeros_like(out_ref)

    x = x_ref[...]
    w1 = w1_ref[0]
    h = jax.lax.dot_general(
        x, w1, (((1,), (1,)), ((), ())),
        preferred_element_type=jnp.float32)
    h = jnp.maximum(h + b1_ref[0], 0.0).astype(jnp.bfloat16)
    w2 = w2_ref[0]
    y = jax.lax.dot_general(
        h, w2, (((1,), (1,)), ((), ())),
        preferred_element_type=jnp.float32)
    y = y + b2_ref[0]
    mask = (idx_ref[...] == e).astype(jnp.float32)
    out_ref[...] += mask * y


@functools.partial(jax.jit, static_argnames=())
def _run(x, emb_idx2d, W1, b1, W2, b2):
    return pl.pallas_call(
        _expert_step,
        grid=(NUM_EMB,),
        in_specs=[
            pl.BlockSpec((B, X_SIZE), lambda e: (0, 0)),
            pl.BlockSpec((B, 1), lambda e: (0, 0)),
            pl.BlockSpec((1, H_SIZE, X_SIZE), lambda e: (e, 0, 0)),
            pl.BlockSpec((1, 1, H_SIZE), lambda e: (e, 0, 0)),
            pl.BlockSpec((1, OUT_SIZE, H_SIZE), lambda e: (e, 0, 0)),
            pl.BlockSpec((1, 1, OUT_SIZE), lambda e: (e, 0, 0)),
        ],
        out_specs=pl.BlockSpec((B, OUT_SIZE), lambda e: (0, 0)),
        out_shape=jax.ShapeDtypeStruct((B, OUT_SIZE), jnp.float32),
    )(x, emb_idx2d, W1, b1, W2, b2)


def kernel(quant_fn, x, emb_idx, W1, b1, W2, b2):
    del quant_fn
    emb_idx2d = emb_idx.reshape(B, 1)
    b1r = b1.reshape(NUM_EMB, 1, H_SIZE)
    b2r = b2.reshape(NUM_EMB, 1, OUT_SIZE)
    return _run(x.astype(jnp.bfloat16), emb_idx2d,
                W1.astype(jnp.bfloat16), b1r,
                W2.astype(jnp.bfloat16), b2r)


# speedup vs baseline: 1.2184x; 1.2184x over previous
import jax
import jax.numpy as jnp
import numpy as np
from jax.experimental import pallas as pl

B = 2048
X_SIZE = 128
H_SIZE = 128
OUT_SIZE = 128
NUM_EMB = 16
XA = X_SIZE + 8
EH = NUM_EMB * H_SIZE


def _fused(x_ref, idx_ref, ecol_ref, w1_ref, w2_ref, b2_ref, out_ref):
    h = jnp.dot(x_ref[...], w1_ref[...],
                preferred_element_type=jnp.float32)
    mask = idx_ref[...] == ecol_ref[...]
    g = jnp.where(mask, jnp.maximum(h, 0.0), 0.0).astype(jnp.bfloat16)
    y = jnp.dot(g, w2_ref[...],
                preferred_element_type=jnp.float32)
    eids = jax.lax.broadcasted_iota(jnp.int32, (B, NUM_EMB), 1)
    onehot = (idx_ref[...] == eids).astype(jnp.bfloat16)
    y = y + jnp.dot(onehot, b2_ref[...],
                    preferred_element_type=jnp.float32)
    out_ref[...] = y


@jax.jit
def _run(x_aug, emb_idx2d, ecol, w1cat, w2cat, b2):
    return pl.pallas_call(
        _fused,
        out_shape=jax.ShapeDtypeStruct((B, OUT_SIZE), jnp.float32),
    )(x_aug, emb_idx2d, ecol, w1cat, w2cat, b2)


def kernel(quant_fn, x, emb_idx, W1, b1, W2, b2):
    del quant_fn
    xb = x.astype(jnp.bfloat16)
    pad = jnp.concatenate(
        [jnp.ones((B, 1), jnp.bfloat16), jnp.zeros((B, 7), jnp.bfloat16)], 1)
    x_aug = jnp.concatenate([xb, pad], axis=1)
    w1cat = jnp.transpose(W1, (2, 0, 1)).reshape(X_SIZE, EH)
    w1cat = jnp.concatenate(
        [w1cat, b1.reshape(1, EH), jnp.zeros((7, EH), jnp.float32)], 0)
    w1cat = w1cat.astype(jnp.bfloat16)
    w2cat = jnp.transpose(W2, (0, 2, 1)).reshape(EH, OUT_SIZE)
    w2cat = w2cat.astype(jnp.bfloat16)
    emb_idx2d = emb_idx.reshape(B, 1)
    ecol = (np.arange(EH, dtype=np.int32) // H_SIZE).reshape(1, EH)
    ecol = jnp.asarray(ecol)
    return _run(x_aug, emb_idx2d, ecol, w1cat, w2cat,
                b2.astype(jnp.bfloat16))


# speedup vs baseline: 1.9617x; 1.6100x over previous
import jax
import jax.numpy as jnp
from jax.experimental import pallas as pl

B = 2048
X_SIZE = 128
H_SIZE = 128
OUT_SIZE = 128
NUM_EMB = 16
EH = NUM_EMB * H_SIZE


def _fused(idx_ref, x_ref, w1_ref, b1_ref, w2_ref, b2_ref, out_ref):
    xb = x_ref[...].astype(jnp.bfloat16)
    w1 = w1_ref[...].astype(jnp.bfloat16)
    h = jax.lax.dot_general(
        xb, w1, (((1,), (1,)), ((), ())),
        preferred_element_type=jnp.float32)
    h = h + b1_ref[...]
    ecol = jax.lax.broadcasted_iota(jnp.int32, (1, EH), 1) // H_SIZE
    mask = idx_ref[...] == ecol
    g = jnp.where(mask, jnp.maximum(h, 0.0), 0.0).astype(jnp.bfloat16)
    w2t = jnp.transpose(w2_ref[...].astype(jnp.bfloat16), (0, 2, 1))
    w2t = w2t.reshape(EH, OUT_SIZE)
    y = jax.lax.dot_general(
        g, w2t, (((1,), (0,)), ((), ())),
        preferred_element_type=jnp.float32)
    eids = jax.lax.broadcasted_iota(jnp.int32, (B, NUM_EMB), 1)
    onehot = (idx_ref[...] == eids).astype(jnp.bfloat16)
    y = y + jax.lax.dot_general(
        onehot, b2_ref[...].astype(jnp.bfloat16), (((1,), (0,)), ((), ())),
        preferred_element_type=jnp.float32)
    out_ref[...] = y


@jax.jit
def _run(emb_idx2d, x, w1flat, b1row, W2, b2):
    return pl.pallas_call(
        _fused,
        out_shape=jax.ShapeDtypeStruct((B, OUT_SIZE), jnp.float32),
    )(emb_idx2d, x, w1flat, b1row, W2, b2)


def kernel(quant_fn, x, emb_idx, W1, b1, W2, b2):
    del quant_fn
    return _run(emb_idx.reshape(B, 1), x,
                W1.reshape(EH, X_SIZE), b1.reshape(1, EH), W2, b2)
